# Initial kernel scaffold; baseline (speedup 1.0000x reference)
import jax
import jax.numpy as jnp
from jax.experimental import pallas as pl


def kernel(X, indexHash, bitHash):
    raise NotImplementedError("write your pallas kernel here")


# speedup vs baseline: 49.8293x; 49.8293x over previous
import functools

import numpy as np
import jax
import jax.numpy as jnp
from jax import lax
from jax.experimental import pallas as pl
from jax.experimental.pallas import tpu as pltpu, tpu_sc as plsc

_N = 4096
_B = 1024
_F = 16384
_KH = 2176
_NW = 32
_RPW = _B // _NW


def _dft_tables():
    n = _N
    k = np.arange(_KH)
    j = np.arange(n)
    valid = (k < n // 2 + 1).astype(np.float64)
    ang = 2.0 * np.pi * (np.outer(j, k) % n) / n
    cf = np.cos(ang) * valid
    sf = -np.sin(ang) * valid
    w1 = np.concatenate([cf, sf], axis=1).astype(np.float32)
    w = np.where((k == 0) | (k == n // 2), 1.0, 2.0) * valid / n
    ang2 = 2.0 * np.pi * (np.outer(k, j) % n) / n
    ai = (w[:, None] * np.cos(ang2)).astype(np.float32)
    di = (-(w[:, None]) * np.sin(ang2)).astype(np.float32)
    return w1, ai, di


_W1_NP, _AI_NP, _DI_NP = _dft_tables()


def _sc_sketch(X, idxc):
    mesh = plsc.VectorSubcoreMesh(core_axis_name="c", subcore_axis_name="s")

    @functools.partial(
        pl.kernel,
        out_type=jax.ShapeDtypeStruct((2, _B, _N), jnp.float32),
        mesh=mesh,
        compiler_params=pltpu.CompilerParams(needs_layout_passes=False),
        scratch_types=[
            pltpu.VMEM((_F,), jnp.int32),
            pltpu.VMEM((_F,), jnp.int32),
            pltpu.VMEM((_F,), jnp.float32),
            pltpu.VMEM((4 * _N,), jnp.float32),
            pltpu.VMEM((_N,), jnp.float32),
            pltpu.VMEM((_N,), jnp.float32),
        ],
    )
    def k(x_hbm, idxc_hbm, s_hbm, idx0_v, idx1_v, x_v, acc_v, o0_v, o1_v):
        wid = lax.axis_index("s") * 2 + lax.axis_index("c")
        pltpu.sync_copy(idxc_hbm.at[0], idx0_v)
        pltpu.sync_copy(idxc_hbm.at[1], idx1_v)
        row0 = wid * _RPW

        def row_body(r, carry):
            row = row0 + r
            pltpu.sync_copy(x_hbm.at[row], x_v)

            def zero_body(i, c):
                acc_v[pl.ds(i * 16, 16)] = jnp.zeros((16,), jnp.float32)
                return c
            lax.fori_loop(0, 4 * _N // 16, zero_body, 0)

            def scat_body(i, c):
                sl = pl.ds(i * 16, 16)
                xv = x_v[sl]
                plsc.addupdate_scatter(acc_v, [idx0_v[sl]], xv)
                plsc.addupdate_scatter(acc_v, [idx1_v[sl]], xv)
                return c
            lax.fori_loop(0, _F // 16, scat_body, 0)

            def out_body(jv, c):
                sl = pl.ds(jv * 16, 16)
                o0_v[sl] = acc_v[pl.ds(jv * 16, 16)] - acc_v[pl.ds(_N + jv * 16, 16)]
                o1_v[sl] = (acc_v[pl.ds(2 * _N + jv * 16, 16)]
                            - acc_v[pl.ds(3 * _N + jv * 16, 16)])
                return c
            lax.fori_loop(0, _N // 16, out_body, 0)

            pltpu.sync_copy(o0_v, s_hbm.at[0, row])
            pltpu.sync_copy(o1_v, s_hbm.at[1, row])
            return carry

        lax.fori_loop(0, _RPW, row_body, 0)

    return k(X, idxc)


def _fwd_matmul(s_in, w1):
    bm, bk, bn = 1024, 512, 2 * _KH
    grid = (2 * _B // bm, 2 * _KH // bn, _N // bk)

    def body(a_ref, b_ref, o_ref):
        @pl.when(pl.program_id(2) == 0)
        def _():
            o_ref[...] = jnp.zeros_like(o_ref)
        o_ref[...] += jnp.dot(a_ref[...], b_ref[...],
                              preferred_element_type=jnp.float32)

    return pl.pallas_call(
        body,
        grid=grid,
        in_specs=[pl.BlockSpec((bm, bk), lambda i, j, k: (i, k)),
                  pl.BlockSpec((bk, bn), lambda i, j, k: (k, j))],
        out_specs=pl.BlockSpec((bm, bn), lambda i, j, k: (i, j)),
        out_shape=jax.ShapeDtypeStruct((2 * _B, 2 * _KH), jnp.float32),
    )(s_in, w1)


def _inv_matmul(o_spec, ai, di):
    bm, bk, bn = 1024, 128, 4096
    kb = _KH // bk
    grid = (_B // bm, _N // bn, kb)

    def body(r0, i0, r1, i1, a_ref, d_ref, o_ref):
        @pl.when(pl.program_id(2) == 0)
        def _():
            o_ref[...] = jnp.zeros_like(o_ref)
        pre = r0[...] * r1[...] - i0[...] * i1[...]
        pim = r0[...] * i1[...] + i0[...] * r1[...]
        o_ref[...] += jnp.dot(pre, a_ref[...], preferred_element_type=jnp.float32)
        o_ref[...] += jnp.dot(pim, d_ref[...], preferred_element_type=jnp.float32)

    in_specs = [
        pl.BlockSpec((bm, bk), lambda i, j, k: (i, k)),
        pl.BlockSpec((bm, bk), lambda i, j, k: (i, k + kb)),
        pl.BlockSpec((bm, bk), lambda i, j, k: (i + 1, k)),
        pl.BlockSpec((bm, bk), lambda i, j, k: (i + 1, k + kb)),
        pl.BlockSpec((bk, bn), lambda i, j, k: (k, j)),
        pl.BlockSpec((bk, bn), lambda i, j, k: (k, j)),
    ]
    return pl.pallas_call(
        body,
        grid=grid,
        in_specs=in_specs,
        out_specs=pl.BlockSpec((bm, bn), lambda i, j, k: (i, j)),
        out_shape=jax.ShapeDtypeStruct((_B, _N), jnp.float32),
    )(o_spec, o_spec, o_spec, o_spec, ai, di)


def kernel(X, indexHash, bitHash):
    offs = jnp.array([[0], [2 * _N]], dtype=jnp.int32)
    idxc = (indexHash + jnp.where(bitHash < 0, _N, 0).astype(jnp.int32) + offs)
    s = _sc_sketch(X, idxc)
    o_spec = _fwd_matmul(s.reshape(2 * _B, _N), jnp.asarray(_W1_NP))
    return _inv_matmul(o_spec, jnp.asarray(_AI_NP), jnp.asarray(_DI_NP))
